# baseline (speedup 1.0000x reference)
import jax
import jax.numpy as jnp
from jax.experimental import pallas as pl


def _conv(x, src, dst, ew, W, b):
    n = x.shape[0]
    loop = jnp.arange(n, dtype=src.dtype)
    src2 = jnp.concatenate([src, loop])
    dst2 = jnp.concatenate([dst, loop])
    ew2 = jnp.concatenate([ew, jnp.ones((n,), dtype=ew.dtype)])
    xw = x @ W
    deg = jax.ops.segment_sum(ew2, dst2, num_segments=n)
    dinv = jnp.where(deg > 0, 1.0 / jnp.sqrt(deg), 0.0)
    norm = dinv[src2] * ew2 * dinv[dst2]
    msgs = xw[src2] * norm[:, None]
    out = jax.ops.segment_sum(msgs, dst2, num_segments=n)
    return out + b


def _l2n(x):
    nrm = jnp.linalg.norm(x, ord=2, axis=1, keepdims=True)
    return x / jnp.maximum(nrm, 1e-12)


def _bias_kernel(h_ref, w_ref, b_ref, o_ref):
    o_ref[...] = h_ref[...] @ w_ref[...] + b_ref[...]


def kernel(x, edge_index, edge_weights, W1, b1, W2, b2, W3, b3, W4, b4, Wl, bl):
    src = edge_index[0]
    dst = edge_index[1]
    h = _conv(x, src, dst, edge_weights, W1, b1)
    h = jax.nn.relu(_l2n(h))
    h = _conv(h, src, dst, edge_weights, W2, b2)
    h = jax.nn.relu(_l2n(h))
    h = _conv(h, src, dst, edge_weights, W3, b3)
    h = jax.nn.relu(_l2n(h))
    h = _conv(h, src, dst, edge_weights, W4, b4)
    h = jax.nn.relu(_l2n(h))
    n = h.shape[0]
    grid = 10
    blk = n // grid
    return pl.pallas_call(
        _bias_kernel,
        grid=(grid,),
        in_specs=[
            pl.BlockSpec((blk, h.shape[1]), lambda i: (i, 0)),
            pl.BlockSpec((h.shape[1], Wl.shape[1]), lambda i: (0, 0)),
            pl.BlockSpec((Wl.shape[1],), lambda i: (0,)),
        ],
        out_specs=pl.BlockSpec((blk, Wl.shape[1]), lambda i: (i, 0)),
        out_shape=jax.ShapeDtypeStruct((n, Wl.shape[1]), jnp.float32),
    )(h, Wl, bl)


# speedup vs baseline: 3.8742x; 3.8742x over previous
import dataclasses

import jax
import jax.numpy as jnp
from jax import lax
from jax.experimental import pallas as pl
from jax.experimental.pallas import tpu as pltpu
from jax.experimental.pallas import tpu_sc as plsc

N = 10000
NP = 10240
E = 320000
F_IN = 128
H = 256
HH = H // 2
C = 40

NSUB = 16
B = 80
ROWS_PER_SUB = NP // NSUB

_mesh = plsc.VectorSubcoreMesh(core_axis_name="c", subcore_axis_name="s")

_cp = pltpu.CompilerParams()
if "needs_layout_passes" in pltpu.CompilerParams.__dataclass_fields__:
    _cp = dataclasses.replace(_cp, needs_layout_passes=False)


def _norm_body(src_hbm, dst_hbm, ew_hbm, dinv_hbm, out_hbm,
               dv_v, sidx_v, didx_v, ew_v, nrm_v):
    c = lax.axis_index("c")
    s = lax.axis_index("s")
    w = s * 2 + c
    pltpu.sync_copy(dinv_hbm, dv_v)

    chunk = E // (2 * NSUB)

    @pl.loop(0, chunk // B)
    def _(t):
        base = w * chunk + t * B
        pltpu.sync_copy(src_hbm.at[pl.ds(base, B)], sidx_v)
        pltpu.sync_copy(dst_hbm.at[pl.ds(base, B)], didx_v)
        pltpu.sync_copy(ew_hbm.at[pl.ds(base, B)], ew_v)
        for j in range(B // 16):
            s16 = sidx_v[pl.ds(j * 16, 16)]
            d16 = didx_v[pl.ds(j * 16, 16)]
            e16 = ew_v[pl.ds(j * 16, 16)]
            g1 = plsc.load_gather(dv_v, [s16])
            g2 = plsc.load_gather(dv_v, [d16])
            nrm_v[pl.ds(j * 16, 16)] = g1 * e16 * g2
        pltpu.sync_copy(nrm_v, out_hbm.at[pl.ds(base, B)])


def _sc_norm(src, dst, ew, dinv):
    k = pl.kernel(
        _norm_body,
        compiler_params=_cp,
        out_type=jax.ShapeDtypeStruct((E,), jnp.float32),
        mesh=_mesh,
        scratch_types=[
            pltpu.VMEM((NP,), jnp.float32),
            pltpu.VMEM((B,), jnp.int32),
            pltpu.VMEM((B,), jnp.int32),
            pltpu.VMEM((B,), jnp.float32),
            pltpu.VMEM((B,), jnp.float32),
        ],
    )
    return k(src, dst, ew, dinv)


def _agg_body(xw_hbm, src_hbm, dst_hbm, nrm_hbm, out_hbm,
              sidx_v, didx_v, nrm_v, rows_v, zb_v, acc_sh):
    c = lax.axis_index("c")
    s = lax.axis_index("s")

    @pl.loop(0, 128)
    def _(i):
        for k in range(8):
            zb_v[i, pl.ds(k * 16, 16)] = jnp.zeros((16,), jnp.float32)

    for j in range(ROWS_PER_SUB // 128):
        pltpu.sync_copy(
            zb_v, acc_sh.at[pl.ds(s * ROWS_PER_SUB + j * 128, 128), :])
    plsc.subcore_barrier()

    chunk = E // NSUB
    roff = jnp.full((16,), c * NP, jnp.int32)

    @pl.loop(0, chunk // B)
    def _(t):
        base = s * chunk + t * B
        pltpu.sync_copy(src_hbm.at[pl.ds(base, B)], sidx_v)
        pltpu.sync_copy(dst_hbm.at[pl.ds(base, B)], didx_v)
        pltpu.sync_copy(nrm_hbm.at[pl.ds(base, B)], nrm_v)
        for j in range(B // 16):
            sidx_v[pl.ds(j * 16, 16)] = sidx_v[pl.ds(j * 16, 16)] + roff
        pltpu.sync_copy(xw_hbm.at[sidx_v], rows_v)

        @pl.loop(0, B)
        def _(i):
            nr = plsc.load_gather(nrm_v, [jnp.full((16,), i, jnp.int32)])
            for k in range(8):
                rows_v[i, pl.ds(k * 16, 16)] = rows_v[i, pl.ds(k * 16, 16)] * nr

        pltpu.sync_copy(rows_v, acc_sh.at[didx_v], add=True)

    plsc.subcore_barrier()
    pltpu.sync_copy(
        acc_sh.at[pl.ds(s * ROWS_PER_SUB, ROWS_PER_SUB), :],
        out_hbm.at[pl.ds(c * NP + s * ROWS_PER_SUB, ROWS_PER_SUB), :],
    )


def _sc_agg(xw_flat, src, dst, nrm):
    k = pl.kernel(
        _agg_body,
        compiler_params=_cp,
        out_type=jax.ShapeDtypeStruct((2 * NP, HH), jnp.float32),
        mesh=_mesh,
        scratch_types=[
            pltpu.VMEM((B,), jnp.int32),
            pltpu.VMEM((B,), jnp.int32),
            pltpu.VMEM((B,), jnp.float32),
            pltpu.VMEM((B, HH), jnp.float32),
            pltpu.VMEM((128, HH), jnp.float32),
            pltpu.VMEM_SHARED((NP, HH), jnp.float32),
        ],
    )
    return k(xw_flat, src, dst, nrm)


_BLK = 1024


def _dot(a, b):
    return lax.dot_general(a, b, (((1,), (0,)), ((), ())),
                           precision=lax.Precision.HIGHEST,
                           preferred_element_type=jnp.float32)


def _first_body(x_ref, w_ref, deg_ref, xw_ref, dinv_ref):
    deg = deg_ref[0, :, 0] + 1.0
    dinv = jnp.where(deg > 0, lax.rsqrt(deg), 0.0)
    dinv_ref[...] = dinv[:, None]
    xw = _dot(x_ref[...], w_ref[...])
    xw_ref[0] = xw[:, :HH]
    xw_ref[1] = xw[:, HH:]


def _tc_first(x, W1, deg2):
    return pl.pallas_call(
        _first_body,
        grid=(NP // _BLK,),
        in_specs=[
            pl.BlockSpec((_BLK, F_IN), lambda i: (i, 0)),
            pl.BlockSpec((F_IN, H), lambda i: (0, 0)),
            pl.BlockSpec((2, _BLK, HH), lambda i: (0, i, 0)),
        ],
        out_specs=[
            pl.BlockSpec((2, _BLK, HH), lambda i: (0, i, 0)),
            pl.BlockSpec((_BLK, 1), lambda i: (i, 0)),
        ],
        out_shape=[
            jax.ShapeDtypeStruct((2, NP, HH), jnp.float32),
            jax.ShapeDtypeStruct((NP, 1), jnp.float32),
        ],
    )(x, W1, deg2)


def _mid_body(agg_ref, xwp_ref, dinv_ref, b_ref, w_ref, out_ref):
    d2 = dinv_ref[...] * dinv_ref[...]
    t = jnp.concatenate(
        [agg_ref[0] + xwp_ref[0] * d2, agg_ref[1] + xwp_ref[1] * d2], axis=1)
    t = t + b_ref[...]
    nrm = jnp.sqrt(jnp.sum(t * t, axis=1, keepdims=True))
    r = t / jnp.maximum(nrm, 1e-12)
    r = jnp.maximum(r, 0.0)
    xw = _dot(r, w_ref[...])
    out_ref[0] = xw[:, :HH]
    out_ref[1] = xw[:, HH:]


def _tc_mid(agg, xwp, dinv, b, Wn):
    return pl.pallas_call(
        _mid_body,
        grid=(NP // _BLK,),
        in_specs=[
            pl.BlockSpec((2, _BLK, HH), lambda i: (0, i, 0)),
            pl.BlockSpec((2, _BLK, HH), lambda i: (0, i, 0)),
            pl.BlockSpec((_BLK, 1), lambda i: (i, 0)),
            pl.BlockSpec((1, H), lambda i: (0, 0)),
            pl.BlockSpec((H, H), lambda i: (0, 0)),
        ],
        out_specs=pl.BlockSpec((2, _BLK, HH), lambda i: (0, i, 0)),
        out_shape=jax.ShapeDtypeStruct((2, NP, HH), jnp.float32),
    )(agg, xwp, dinv, b, Wn)


def _last_body(agg_ref, xwp_ref, dinv_ref, b_ref, wl_ref, bl_ref, out_ref):
    d2 = dinv_ref[...] * dinv_ref[...]
    t = jnp.concatenate(
        [agg_ref[0] + xwp_ref[0] * d2, agg_ref[1] + xwp_ref[1] * d2], axis=1)
    t = t + b_ref[...]
    nrm = jnp.sqrt(jnp.sum(t * t, axis=1, keepdims=True))
    r = t / jnp.maximum(nrm, 1e-12)
    r = jnp.maximum(r, 0.0)
    out_ref[...] = _dot(r, wl_ref[...]) + bl_ref[...]


def _tc_last(agg, xwp, dinv, b, Wl, bl):
    return pl.pallas_call(
        _last_body,
        grid=(NP // _BLK,),
        in_specs=[
            pl.BlockSpec((2, _BLK, HH), lambda i: (0, i, 0)),
            pl.BlockSpec((2, _BLK, HH), lambda i: (0, i, 0)),
            pl.BlockSpec((_BLK, 1), lambda i: (i, 0)),
            pl.BlockSpec((1, H), lambda i: (0, 0)),
            pl.BlockSpec((H, C), lambda i: (0, 0)),
            pl.BlockSpec((1, C), lambda i: (0, 0)),
        ],
        out_specs=pl.BlockSpec((_BLK, C), lambda i: (i, 0)),
        out_shape=jax.ShapeDtypeStruct((NP, C), jnp.float32),
    )(agg, xwp, dinv, b, Wl, bl)


_DBG_SC_DEG = True
_DBG_SC_NORM = True
_DBG_SC_AGG = True


def _xla_deg2(dst, ew):
    deg = jax.ops.segment_sum(ew, dst, num_segments=NP)
    z = jnp.zeros((NP, HH), jnp.float32)
    return jnp.stack([deg[:, None] + z, z])


def _xla_norm(src, dst, ew, dinv):
    d = dinv.reshape(NP)
    return d[src] * ew * d[dst]


def _xla_agg(xw, src, dst, nrm):
    full = jnp.concatenate([xw[0], xw[1]], axis=1)
    msgs = full[src] * nrm[:, None]
    out = jax.ops.segment_sum(msgs, dst, num_segments=NP)
    return jnp.stack([out[:, :HH], out[:, HH:]])


def kernel(x, edge_index, edge_weights, W1, b1, W2, b2, W3, b3, W4, b4, Wl, bl):
    src = edge_index[0]
    dst = edge_index[1]
    ew = edge_weights.astype(jnp.float32)

    if _DBG_SC_DEG:
        ones = jnp.ones((2 * NP, HH), jnp.float32)
        deg2 = _sc_agg(ones, src, dst, ew).reshape(2, NP, HH)
    else:
        deg2 = _xla_deg2(dst, ew)
    xp = jnp.pad(x, ((0, NP - N), (0, 0)))
    xw, dinv = _tc_first(xp, W1, deg2)
    if _DBG_SC_NORM:
        nrm = _sc_norm(src, dst, ew, dinv.reshape(NP))
    else:
        nrm = _xla_norm(src, dst, ew, dinv)

    biases = [b1.reshape(1, H), b2.reshape(1, H), b3.reshape(1, H),
              b4.reshape(1, H)]
    weights = [W2, W3, W4]

    def _agg(xw):
        if _DBG_SC_AGG:
            return _sc_agg(xw.reshape(2 * NP, HH), src, dst,
                           nrm).reshape(2, NP, HH)
        return _xla_agg(xw, src, dst, nrm)

    for l in range(3):
        agg = _agg(xw)
        xw = _tc_mid(agg, xw, dinv, biases[l], weights[l])

    agg = _agg(xw)
    return _tc_last(agg, xw, dinv, biases[3], Wl, bl.reshape(1, C))[:N]


# speedup vs baseline: 6.8602x; 1.7707x over previous
import dataclasses

import jax
import jax.numpy as jnp
from jax import lax
from jax.experimental import pallas as pl
from jax.experimental.pallas import tpu as pltpu
from jax.experimental.pallas import tpu_sc as plsc

N = 10000
NP = 10240
E = 320000
F_IN = 128
H = 256
HH = H // 2
C = 40

NSUB = 16
B = 80
ROWS_PER_SUB = NP // NSUB

_mesh = plsc.VectorSubcoreMesh(core_axis_name="c", subcore_axis_name="s")

_cp = pltpu.CompilerParams()
if "needs_layout_passes" in pltpu.CompilerParams.__dataclass_fields__:
    _cp = dataclasses.replace(_cp, needs_layout_passes=False)


def _norm_body(src_hbm, dst_hbm, ew_hbm, dinv_hbm, out_hbm,
               dv_v, sidx_v, didx_v, ew_v, nrm_v):
    c = lax.axis_index("c")
    s = lax.axis_index("s")
    w = s * 2 + c
    pltpu.sync_copy(dinv_hbm, dv_v)

    chunk = E // (2 * NSUB)

    @pl.loop(0, chunk // B)
    def _(t):
        base = w * chunk + t * B
        pltpu.sync_copy(src_hbm.at[pl.ds(base, B)], sidx_v)
        pltpu.sync_copy(dst_hbm.at[pl.ds(base, B)], didx_v)
        pltpu.sync_copy(ew_hbm.at[pl.ds(base, B)], ew_v)
        for j in range(B // 16):
            s16 = sidx_v[pl.ds(j * 16, 16)]
            d16 = didx_v[pl.ds(j * 16, 16)]
            e16 = ew_v[pl.ds(j * 16, 16)]
            g1 = plsc.load_gather(dv_v, [s16])
            g2 = plsc.load_gather(dv_v, [d16])
            nrm_v[pl.ds(j * 16, 16)] = g1 * e16 * g2
        pltpu.sync_copy(nrm_v, out_hbm.at[pl.ds(base, B)])


def _sc_norm(src, dst, ew, dinv):
    k = pl.kernel(
        _norm_body,
        compiler_params=_cp,
        out_type=jax.ShapeDtypeStruct((E,), jnp.float32),
        mesh=_mesh,
        scratch_types=[
            pltpu.VMEM((NP,), jnp.float32),
            pltpu.VMEM((B,), jnp.int32),
            pltpu.VMEM((B,), jnp.int32),
            pltpu.VMEM((B,), jnp.float32),
            pltpu.VMEM((B,), jnp.float32),
        ],
    )
    return k(src, dst, ew, dinv)


NBUF = 2


def _agg_body(xw_hbm, src_hbm, dst_hbm, nrm_hbm, out_hbm,
              sidx_bufs, didx_bufs, nrm_bufs, rows_bufs, zb_v, acc_sh,
              isems, gsems, dsems, nsems, ssems):
    c = lax.axis_index("c")
    s = lax.axis_index("s")

    @pl.loop(0, 64)
    def _(i):
        for k in range(8):
            zb_v[i, pl.ds(k * 16, 16)] = jnp.zeros((16,), jnp.float32)

    for j in range(ROWS_PER_SUB // 64):
        pltpu.sync_copy(
            zb_v, acc_sh.at[pl.ds(s * ROWS_PER_SUB + j * 64, 64), :])
    plsc.subcore_barrier()

    chunk = E // NSUB
    cbase = s * chunk
    roff = jnp.full((16,), c * NP, jnp.int32)

    @pl.loop(0, chunk // (B * NBUF))
    def _(a):
        base = cbase + a * (B * NBUF)
        iw = []
        dw = []
        nw = []
        for b in range(NBUF):
            bb = base + b * B
            iw.append(pltpu.async_copy(
                src_hbm.at[pl.ds(bb, B)], sidx_bufs[b], isems[b]))
            dw.append(pltpu.async_copy(
                dst_hbm.at[pl.ds(bb, B)], didx_bufs[b], dsems[b]))
            nw.append(pltpu.async_copy(
                nrm_hbm.at[pl.ds(bb, B)], nrm_bufs[b], nsems[b]))
        gw = []
        for b in range(NBUF):
            iw[b].wait()
            sidx_v = sidx_bufs[b]
            for j in range(B // 16):
                sl = pl.ds(j * 16, 16)
                sidx_v[sl] = sidx_v[sl] + roff
            gw.append(pltpu.async_copy(
                xw_hbm.at[sidx_v], rows_bufs[b], gsems[b]))
        sw = []
        for b in range(NBUF):
            gw[b].wait()
            nw[b].wait()
            rows_v = rows_bufs[b]
            nrm_v = nrm_bufs[b]

            @pl.loop(0, B, step=2)
            def _(i):
                for u in range(2):
                    nr = plsc.load_gather(
                        nrm_v, [jnp.full((16,), i + u, jnp.int32)])
                    for k in range(8):
                        sl = (i + u, pl.ds(k * 16, 16))
                        rows_v[sl] = rows_v[sl] * nr

            dw[b].wait()
            sw.append(pltpu.async_copy(
                rows_v, acc_sh.at[didx_bufs[b]], ssems[b], add=True))
        for b in range(NBUF):
            sw[b].wait()

    plsc.subcore_barrier()
    pltpu.sync_copy(
        acc_sh.at[pl.ds(s * ROWS_PER_SUB, ROWS_PER_SUB), :],
        out_hbm.at[pl.ds(c * NP + s * ROWS_PER_SUB, ROWS_PER_SUB), :],
    )


def _sc_agg(xw_flat, src, dst, nrm):
    k = pl.kernel(
        _agg_body,
        compiler_params=_cp,
        out_type=jax.ShapeDtypeStruct((2 * NP, HH), jnp.float32),
        mesh=_mesh,
        scratch_types=[
            [pltpu.VMEM((B,), jnp.int32) for _ in range(NBUF)],
            [pltpu.VMEM((B,), jnp.int32) for _ in range(NBUF)],
            [pltpu.VMEM((B,), jnp.float32) for _ in range(NBUF)],
            [pltpu.VMEM((B, HH), jnp.float32) for _ in range(NBUF)],
            pltpu.VMEM((64, HH), jnp.float32),
            pltpu.VMEM_SHARED((NP, HH), jnp.float32),
            [pltpu.SemaphoreType.DMA for _ in range(NBUF)],
            [pltpu.SemaphoreType.DMA for _ in range(NBUF)],
            [pltpu.SemaphoreType.DMA for _ in range(NBUF)],
            [pltpu.SemaphoreType.DMA for _ in range(NBUF)],
            [pltpu.SemaphoreType.DMA for _ in range(NBUF)],
        ],
    )
    return k(xw_flat, src, dst, nrm)


_BLK = 1024


def _dot(a, b):
    return lax.dot_general(a, b, (((1,), (0,)), ((), ())),
                           precision=lax.Precision.HIGHEST,
                           preferred_element_type=jnp.float32)


def _first_body(x_ref, w_ref, deg_ref, xw_ref, dinv_ref):
    deg = deg_ref[0, :, 0] + 1.0
    dinv = jnp.where(deg > 0, lax.rsqrt(deg), 0.0)
    dinv_ref[...] = dinv[:, None]
    xw = _dot(x_ref[...], w_ref[...])
    xw_ref[0] = xw[:, :HH]
    xw_ref[1] = xw[:, HH:]


def _tc_first(x, W1, deg2):
    return pl.pallas_call(
        _first_body,
        grid=(NP // _BLK,),
        in_specs=[
            pl.BlockSpec((_BLK, F_IN), lambda i: (i, 0)),
            pl.BlockSpec((F_IN, H), lambda i: (0, 0)),
            pl.BlockSpec((2, _BLK, HH), lambda i: (0, i, 0)),
        ],
        out_specs=[
            pl.BlockSpec((2, _BLK, HH), lambda i: (0, i, 0)),
            pl.BlockSpec((_BLK, 1), lambda i: (i, 0)),
        ],
        out_shape=[
            jax.ShapeDtypeStruct((2, NP, HH), jnp.float32),
            jax.ShapeDtypeStruct((NP, 1), jnp.float32),
        ],
    )(x, W1, deg2)


def _mid_body(agg_ref, xwp_ref, dinv_ref, b_ref, w_ref, out_ref):
    d2 = dinv_ref[...] * dinv_ref[...]
    t = jnp.concatenate(
        [agg_ref[0] + xwp_ref[0] * d2, agg_ref[1] + xwp_ref[1] * d2], axis=1)
    t = t + b_ref[...]
    nrm = jnp.sqrt(jnp.sum(t * t, axis=1, keepdims=True))
    r = t / jnp.maximum(nrm, 1e-12)
    r = jnp.maximum(r, 0.0)
    xw = _dot(r, w_ref[...])
    out_ref[0] = xw[:, :HH]
    out_ref[1] = xw[:, HH:]


def _tc_mid(agg, xwp, dinv, b, Wn):
    return pl.pallas_call(
        _mid_body,
        grid=(NP // _BLK,),
        in_specs=[
            pl.BlockSpec((2, _BLK, HH), lambda i: (0, i, 0)),
            pl.BlockSpec((2, _BLK, HH), lambda i: (0, i, 0)),
            pl.BlockSpec((_BLK, 1), lambda i: (i, 0)),
            pl.BlockSpec((1, H), lambda i: (0, 0)),
            pl.BlockSpec((H, H), lambda i: (0, 0)),
        ],
        out_specs=pl.BlockSpec((2, _BLK, HH), lambda i: (0, i, 0)),
        out_shape=jax.ShapeDtypeStruct((2, NP, HH), jnp.float32),
    )(agg, xwp, dinv, b, Wn)


def _last_body(agg_ref, xwp_ref, dinv_ref, b_ref, wl_ref, bl_ref, out_ref):
    d2 = dinv_ref[...] * dinv_ref[...]
    t = jnp.concatenate(
        [agg_ref[0] + xwp_ref[0] * d2, agg_ref[1] + xwp_ref[1] * d2], axis=1)
    t = t + b_ref[...]
    nrm = jnp.sqrt(jnp.sum(t * t, axis=1, keepdims=True))
    r = t / jnp.maximum(nrm, 1e-12)
    r = jnp.maximum(r, 0.0)
    out_ref[...] = _dot(r, wl_ref[...]) + bl_ref[...]


def _tc_last(agg, xwp, dinv, b, Wl, bl):
    return pl.pallas_call(
        _last_body,
        grid=(NP // _BLK,),
        in_specs=[
            pl.BlockSpec((2, _BLK, HH), lambda i: (0, i, 0)),
            pl.BlockSpec((2, _BLK, HH), lambda i: (0, i, 0)),
            pl.BlockSpec((_BLK, 1), lambda i: (i, 0)),
            pl.BlockSpec((1, H), lambda i: (0, 0)),
            pl.BlockSpec((H, C), lambda i: (0, 0)),
            pl.BlockSpec((1, C), lambda i: (0, 0)),
        ],
        out_specs=pl.BlockSpec((_BLK, C), lambda i: (i, 0)),
        out_shape=jax.ShapeDtypeStruct((NP, C), jnp.float32),
    )(agg, xwp, dinv, b, Wl, bl)


_DBG_SC_DEG = True
_DBG_SC_NORM = True
_DBG_SC_AGG = True


def _xla_deg2(dst, ew):
    deg = jax.ops.segment_sum(ew, dst, num_segments=NP)
    z = jnp.zeros((NP, HH), jnp.float32)
    return jnp.stack([deg[:, None] + z, z])


def _xla_norm(src, dst, ew, dinv):
    d = dinv.reshape(NP)
    return d[src] * ew * d[dst]


def _xla_agg(xw, src, dst, nrm):
    full = jnp.concatenate([xw[0], xw[1]], axis=1)
    msgs = full[src] * nrm[:, None]
    out = jax.ops.segment_sum(msgs, dst, num_segments=NP)
    return jnp.stack([out[:, :HH], out[:, HH:]])


def kernel(x, edge_index, edge_weights, W1, b1, W2, b2, W3, b3, W4, b4, Wl, bl):
    src = edge_index[0]
    dst = edge_index[1]
    ew = edge_weights.astype(jnp.float32)

    if _DBG_SC_DEG:
        ones = jnp.ones((2 * NP, HH), jnp.float32)
        deg2 = _sc_agg(ones, src, dst, ew).reshape(2, NP, HH)
    else:
        deg2 = _xla_deg2(dst, ew)
    xp = jnp.pad(x, ((0, NP - N), (0, 0)))
    xw, dinv = _tc_first(xp, W1, deg2)
    if _DBG_SC_NORM:
        nrm = _sc_norm(src, dst, ew, dinv.reshape(NP))
    else:
        nrm = _xla_norm(src, dst, ew, dinv)

    biases = [b1.reshape(1, H), b2.reshape(1, H), b3.reshape(1, H),
              b4.reshape(1, H)]
    weights = [W2, W3, W4]

    def _agg(xw):
        if _DBG_SC_AGG:
            return _sc_agg(xw.reshape(2 * NP, HH), src, dst,
                           nrm).reshape(2, NP, HH)
        return _xla_agg(xw, src, dst, nrm)

    for l in range(3):
        agg = _agg(xw)
        xw = _tc_mid(agg, xw, dinv, biases[l], weights[l])

    agg = _agg(xw)
    return _tc_last(agg, xw, dinv, biases[3], Wl, bl.reshape(1, C))[:N]


# speedup vs baseline: 8.6389x; 1.2593x over previous
import dataclasses

import jax
import jax.numpy as jnp
from jax import lax
from jax.experimental import pallas as pl
from jax.experimental.pallas import tpu as pltpu
from jax.experimental.pallas import tpu_sc as plsc

N = 10000
NP = 10240
E = 320000
F_IN = 128
H = 256
HH = H // 2
C = 40

NSUB = 16
B = 128
BT = 32
ROWS_PER_SUB = NP // NSUB

_mesh = plsc.VectorSubcoreMesh(core_axis_name="c", subcore_axis_name="s")

_cp = pltpu.CompilerParams()
if "needs_layout_passes" in pltpu.CompilerParams.__dataclass_fields__:
    _cp = dataclasses.replace(_cp, needs_layout_passes=False)


def _deg_body(dst_hbm, ew_hbm, out_hbm,
              didx_bufs, ew_bufs, rows_bufs, tdidx_v, zb_v, acc_sh,
              dsems, esems, ssems):
    c = lax.axis_index("c")
    s = lax.axis_index("s")

    @pl.loop(0, 64)
    def _(i):
        for k in range(8):
            zb_v[i, pl.ds(k * 16, 16)] = jnp.zeros((16,), jnp.float32)

    for j in range(ROWS_PER_SUB // 64):
        pltpu.sync_copy(
            zb_v, acc_sh.at[pl.ds(s * ROWS_PER_SUB + j * 64, 64), :])
    plsc.subcore_barrier()

    chunk = E // (2 * NSUB)
    cbase = c * (E // 2) + s * chunk
    nmain = (chunk // B) // NBUF * NBUF

    @pl.loop(0, nmain // NBUF)
    def _(a):
        base = cbase + a * (B * NBUF)
        dw = []
        ewt = []
        for b in range(NBUF):
            bb = base + b * B
            dw.append(pltpu.async_copy(
                dst_hbm.at[pl.ds(bb, B)], didx_bufs[b], dsems[b]))
            ewt.append(pltpu.async_copy(
                ew_hbm.at[pl.ds(bb, B)], ew_bufs[b], esems[b]))
        sw = []
        for b in range(NBUF):
            ewt[b].wait()
            rows_v = rows_bufs[b]
            ew_v = ew_bufs[b]

            @pl.loop(0, B, step=2)
            def _(i):
                for u in range(2):
                    nr = plsc.load_gather(
                        ew_v, [jnp.full((16,), i + u, jnp.int32)])
                    for k in range(8):
                        rows_v[i + u, pl.ds(k * 16, 16)] = nr

            dw[b].wait()
            sw.append(pltpu.async_copy(
                rows_v, acc_sh.at[didx_bufs[b]], ssems[b], add=True))
        for b in range(NBUF):
            sw[b].wait()

    tb = cbase + nmain * B
    tn = chunk - nmain * B
    pltpu.sync_copy(dst_hbm.at[pl.ds(tb, tn)], tdidx_v)
    pltpu.sync_copy(ew_hbm.at[pl.ds(tb, tn)], ew_bufs[0].at[pl.ds(0, tn)])

    @pl.loop(0, tn)
    def _(i):
        nr = plsc.load_gather(ew_bufs[0], [jnp.full((16,), i, jnp.int32)])
        for k in range(8):
            rows_bufs[0][i, pl.ds(k * 16, 16)] = nr

    pltpu.sync_copy(rows_bufs[0].at[pl.ds(0, tn), :],
                    acc_sh.at[tdidx_v], add=True)

    plsc.subcore_barrier()
    pltpu.sync_copy(
        acc_sh.at[pl.ds(s * ROWS_PER_SUB, ROWS_PER_SUB), :],
        out_hbm.at[pl.ds(c * NP + s * ROWS_PER_SUB, ROWS_PER_SUB), :],
    )


def _sc_deg(dst, ew):
    k = pl.kernel(
        _deg_body,
        compiler_params=_cp,
        out_type=jax.ShapeDtypeStruct((2 * NP, HH), jnp.float32),
        mesh=_mesh,
        scratch_types=[
            [pltpu.VMEM((B,), jnp.int32) for _ in range(NBUF)],
            [pltpu.VMEM((B,), jnp.float32) for _ in range(NBUF)],
            [pltpu.VMEM((B, HH), jnp.float32) for _ in range(NBUF)],
            pltpu.VMEM((E // (2 * NSUB) - (E // (2 * NSUB)) // B // NBUF * NBUF * B,), jnp.int32),
            pltpu.VMEM((64, HH), jnp.float32),
            pltpu.VMEM_SHARED((NP, HH), jnp.float32),
            [pltpu.SemaphoreType.DMA for _ in range(NBUF)],
            [pltpu.SemaphoreType.DMA for _ in range(NBUF)],
            [pltpu.SemaphoreType.DMA for _ in range(NBUF)],
        ],
    )
    return k(dst, ew)


def _norm_body(src_hbm, dst_hbm, ew_hbm, dinv_hbm, out_hbm,
               dv_v, sidx_v, didx_v, ew_v, nrm_v):
    c = lax.axis_index("c")
    s = lax.axis_index("s")
    w = s * 2 + c
    pltpu.sync_copy(dinv_hbm, dv_v)

    chunk = E // (2 * NSUB)

    @pl.loop(0, chunk // B)
    def _(t):
        base = w * chunk + t * B
        pltpu.sync_copy(src_hbm.at[pl.ds(base, B)], sidx_v)
        pltpu.sync_copy(dst_hbm.at[pl.ds(base, B)], didx_v)
        pltpu.sync_copy(ew_hbm.at[pl.ds(base, B)], ew_v)
        for j in range(B // 16):
            s16 = sidx_v[pl.ds(j * 16, 16)]
            d16 = didx_v[pl.ds(j * 16, 16)]
            e16 = ew_v[pl.ds(j * 16, 16)]
            g1 = plsc.load_gather(dv_v, [s16])
            g2 = plsc.load_gather(dv_v, [d16])
            nrm_v[pl.ds(j * 16, 16)] = g1 * e16 * g2
        pltpu.sync_copy(nrm_v, out_hbm.at[pl.ds(base, B)])

    nmain = chunk // B
    tn = chunk - nmain * B
    tb = w * chunk + nmain * B
    pltpu.sync_copy(src_hbm.at[pl.ds(tb, tn)], sidx_v.at[pl.ds(0, tn)])
    pltpu.sync_copy(dst_hbm.at[pl.ds(tb, tn)], didx_v.at[pl.ds(0, tn)])
    pltpu.sync_copy(ew_hbm.at[pl.ds(tb, tn)], ew_v.at[pl.ds(0, tn)])
    for j in range(tn // 16):
        s16 = sidx_v[pl.ds(j * 16, 16)]
        d16 = didx_v[pl.ds(j * 16, 16)]
        e16 = ew_v[pl.ds(j * 16, 16)]
        g1 = plsc.load_gather(dv_v, [s16])
        g2 = plsc.load_gather(dv_v, [d16])
        nrm_v[pl.ds(j * 16, 16)] = g1 * e16 * g2
    pltpu.sync_copy(nrm_v.at[pl.ds(0, tn)], out_hbm.at[pl.ds(tb, tn)])


def _sc_norm(src, dst, ew, dinv):
    k = pl.kernel(
        _norm_body,
        compiler_params=_cp,
        out_type=jax.ShapeDtypeStruct((E,), jnp.float32),
        mesh=_mesh,
        scratch_types=[
            pltpu.VMEM((NP,), jnp.float32),
            pltpu.VMEM((B,), jnp.int32),
            pltpu.VMEM((B,), jnp.int32),
            pltpu.VMEM((B,), jnp.float32),
            pltpu.VMEM((B,), jnp.float32),
        ],
    )
    return k(src, dst, ew, dinv)


NBUF = 2


def _agg_body(xw_hbm, src_hbm, dst_hbm, nrm_hbm, out_hbm,
              sidx_bufs, didx_bufs, nrm_bufs, rows_bufs, tdidx_v, zb_v, acc_sh,
              isems, gsems, dsems, nsems, ssems):
    c = lax.axis_index("c")
    s = lax.axis_index("s")

    @pl.loop(0, 64)
    def _(i):
        for k in range(8):
            zb_v[i, pl.ds(k * 16, 16)] = jnp.zeros((16,), jnp.float32)

    for j in range(ROWS_PER_SUB // 64):
        pltpu.sync_copy(
            zb_v, acc_sh.at[pl.ds(s * ROWS_PER_SUB + j * 64, 64), :])
    plsc.subcore_barrier()

    chunk = E // NSUB
    cbase = s * chunk
    roff = jnp.full((16,), c * NP, jnp.int32)

    nmain = (chunk // B) // NBUF * NBUF

    @pl.loop(0, nmain // NBUF)
    def _(a):
        base = cbase + a * (B * NBUF)
        iw = []
        dw = []
        nw = []
        for b in range(NBUF):
            bb = base + b * B
            iw.append(pltpu.async_copy(
                src_hbm.at[pl.ds(bb, B)], sidx_bufs[b], isems[b]))
            dw.append(pltpu.async_copy(
                dst_hbm.at[pl.ds(bb, B)], didx_bufs[b], dsems[b]))
            nw.append(pltpu.async_copy(
                nrm_hbm.at[pl.ds(bb, B)], nrm_bufs[b], nsems[b]))
        gw = []
        for b in range(NBUF):
            iw[b].wait()
            sidx_v = sidx_bufs[b]
            for j in range(B // 16):
                sl = pl.ds(j * 16, 16)
                sidx_v[sl] = sidx_v[sl] + roff
            gw.append(pltpu.async_copy(
                xw_hbm.at[sidx_v], rows_bufs[b], gsems[b]))
        sw = []
        for b in range(NBUF):
            gw[b].wait()
            nw[b].wait()
            rows_v = rows_bufs[b]
            nrm_v = nrm_bufs[b]

            @pl.loop(0, B, step=2)
            def _(i):
                for u in range(2):
                    nr = plsc.load_gather(
                        nrm_v, [jnp.full((16,), i + u, jnp.int32)])
                    for k in range(8):
                        sl = (i + u, pl.ds(k * 16, 16))
                        rows_v[sl] = rows_v[sl] * nr

            dw[b].wait()
            sw.append(pltpu.async_copy(
                rows_v, acc_sh.at[didx_bufs[b]], ssems[b], add=True))
        for b in range(NBUF):
            sw[b].wait()

    tb = cbase + nmain * B
    pltpu.sync_copy(src_hbm.at[pl.ds(tb, BT)], sidx_bufs[0].at[pl.ds(0, BT)])
    pltpu.sync_copy(dst_hbm.at[pl.ds(tb, BT)], tdidx_v)
    pltpu.sync_copy(nrm_hbm.at[pl.ds(tb, BT)], nrm_bufs[0].at[pl.ds(0, BT)])
    for j in range(BT // 16):
        sl = pl.ds(j * 16, 16)
        sidx_bufs[0][sl] = sidx_bufs[0][sl] + roff
    pltpu.sync_copy(xw_hbm.at[sidx_bufs[0].at[pl.ds(0, BT)]],
                    rows_bufs[0].at[pl.ds(0, BT), :])

    @pl.loop(0, BT)
    def _(i):
        nr = plsc.load_gather(nrm_bufs[0], [jnp.full((16,), i, jnp.int32)])
        for k in range(8):
            sl = (i, pl.ds(k * 16, 16))
            rows_bufs[0][sl] = rows_bufs[0][sl] * nr

    pltpu.sync_copy(rows_bufs[0].at[pl.ds(0, BT), :],
                    acc_sh.at[tdidx_v], add=True)

    plsc.subcore_barrier()
    pltpu.sync_copy(
        acc_sh.at[pl.ds(s * ROWS_PER_SUB, ROWS_PER_SUB), :],
        out_hbm.at[pl.ds(c * NP + s * ROWS_PER_SUB, ROWS_PER_SUB), :],
    )


def _sc_agg(xw_flat, src, dst, nrm):
    k = pl.kernel(
        _agg_body,
        compiler_params=_cp,
        out_type=jax.ShapeDtypeStruct((2 * NP, HH), jnp.float32),
        mesh=_mesh,
        scratch_types=[
            [pltpu.VMEM((B,), jnp.int32) for _ in range(NBUF)],
            [pltpu.VMEM((B,), jnp.int32) for _ in range(NBUF)],
            [pltpu.VMEM((B,), jnp.float32) for _ in range(NBUF)],
            [pltpu.VMEM((B, HH), jnp.float32) for _ in range(NBUF)],
            pltpu.VMEM((BT,), jnp.int32),
            pltpu.VMEM((64, HH), jnp.float32),
            pltpu.VMEM_SHARED((NP, HH), jnp.float32),
            [pltpu.SemaphoreType.DMA for _ in range(NBUF)],
            [pltpu.SemaphoreType.DMA for _ in range(NBUF)],
            [pltpu.SemaphoreType.DMA for _ in range(NBUF)],
            [pltpu.SemaphoreType.DMA for _ in range(NBUF)],
            [pltpu.SemaphoreType.DMA for _ in range(NBUF)],
        ],
    )
    return k(xw_flat, src, dst, nrm)


_BLK = 1024


def _dot(a, b):
    return lax.dot_general(a, b, (((1,), (0,)), ((), ())),
                           precision=lax.Precision.HIGHEST,
                           preferred_element_type=jnp.float32)


def _first_body(x_ref, w_ref, deg_ref, xw_ref, dinv_ref):
    deg = deg_ref[0, :, 0] + deg_ref[1, :, 0] + 1.0
    dinv = jnp.where(deg > 0, lax.rsqrt(deg), 0.0)
    dinv_ref[...] = dinv[:, None]
    xw = _dot(x_ref[...], w_ref[...])
    xw_ref[0] = xw[:, :HH]
    xw_ref[1] = xw[:, HH:]


def _tc_first(x, W1, deg2):
    return pl.pallas_call(
        _first_body,
        grid=(NP // _BLK,),
        in_specs=[
            pl.BlockSpec((_BLK, F_IN), lambda i: (i, 0)),
            pl.BlockSpec((F_IN, H), lambda i: (0, 0)),
            pl.BlockSpec((2, _BLK, HH), lambda i: (0, i, 0)),
        ],
        out_specs=[
            pl.BlockSpec((2, _BLK, HH), lambda i: (0, i, 0)),
            pl.BlockSpec((_BLK, 1), lambda i: (i, 0)),
        ],
        out_shape=[
            jax.ShapeDtypeStruct((2, NP, HH), jnp.float32),
            jax.ShapeDtypeStruct((NP, 1), jnp.float32),
        ],
    )(x, W1, deg2)


def _mid_body(agg_ref, xwp_ref, dinv_ref, b_ref, w_ref, out_ref):
    d2 = dinv_ref[...] * dinv_ref[...]
    t = jnp.concatenate(
        [agg_ref[0] + xwp_ref[0] * d2, agg_ref[1] + xwp_ref[1] * d2], axis=1)
    t = t + b_ref[...]
    nrm = jnp.sqrt(jnp.sum(t * t, axis=1, keepdims=True))
    r = t / jnp.maximum(nrm, 1e-12)
    r = jnp.maximum(r, 0.0)
    xw = _dot(r, w_ref[...])
    out_ref[0] = xw[:, :HH]
    out_ref[1] = xw[:, HH:]


def _tc_mid(agg, xwp, dinv, b, Wn):
    return pl.pallas_call(
        _mid_body,
        grid=(NP // _BLK,),
        in_specs=[
            pl.BlockSpec((2, _BLK, HH), lambda i: (0, i, 0)),
            pl.BlockSpec((2, _BLK, HH), lambda i: (0, i, 0)),
            pl.BlockSpec((_BLK, 1), lambda i: (i, 0)),
            pl.BlockSpec((1, H), lambda i: (0, 0)),
            pl.BlockSpec((H, H), lambda i: (0, 0)),
        ],
        out_specs=pl.BlockSpec((2, _BLK, HH), lambda i: (0, i, 0)),
        out_shape=jax.ShapeDtypeStruct((2, NP, HH), jnp.float32),
    )(agg, xwp, dinv, b, Wn)


def _last_body(agg_ref, xwp_ref, dinv_ref, b_ref, wl_ref, bl_ref, out_ref):
    d2 = dinv_ref[...] * dinv_ref[...]
    t = jnp.concatenate(
        [agg_ref[0] + xwp_ref[0] * d2, agg_ref[1] + xwp_ref[1] * d2], axis=1)
    t = t + b_ref[...]
    nrm = jnp.sqrt(jnp.sum(t * t, axis=1, keepdims=True))
    r = t / jnp.maximum(nrm, 1e-12)
    r = jnp.maximum(r, 0.0)
    out_ref[...] = _dot(r, wl_ref[...]) + bl_ref[...]


def _tc_last(agg, xwp, dinv, b, Wl, bl):
    return pl.pallas_call(
        _last_body,
        grid=(NP // _BLK,),
        in_specs=[
            pl.BlockSpec((2, _BLK, HH), lambda i: (0, i, 0)),
            pl.BlockSpec((2, _BLK, HH), lambda i: (0, i, 0)),
            pl.BlockSpec((_BLK, 1), lambda i: (i, 0)),
            pl.BlockSpec((1, H), lambda i: (0, 0)),
            pl.BlockSpec((H, C), lambda i: (0, 0)),
            pl.BlockSpec((1, C), lambda i: (0, 0)),
        ],
        out_specs=pl.BlockSpec((_BLK, C), lambda i: (i, 0)),
        out_shape=jax.ShapeDtypeStruct((NP, C), jnp.float32),
    )(agg, xwp, dinv, b, Wl, bl)


def kernel(x, edge_index, edge_weights, W1, b1, W2, b2, W3, b3, W4, b4, Wl, bl):
    src = edge_index[0]
    dst = edge_index[1]
    ew = edge_weights.astype(jnp.float32)

    deg2 = _sc_deg(dst, ew).reshape(2, NP, HH)
    xp = jnp.pad(x, ((0, NP - N), (0, 0)))
    xw, dinv = _tc_first(xp, W1, deg2)
    nrm = _sc_norm(src, dst, ew, dinv.reshape(NP))

    biases = [b1.reshape(1, H), b2.reshape(1, H), b3.reshape(1, H),
              b4.reshape(1, H)]
    weights = [W2, W3, W4]

    for l in range(3):
        agg = _sc_agg(xw.reshape(2 * NP, HH), src, dst, nrm).reshape(2, NP, HH)
        xw = _tc_mid(agg, xw, dinv, biases[l], weights[l])

    agg = _sc_agg(xw.reshape(2 * NP, HH), src, dst, nrm).reshape(2, NP, HH)
    return _tc_last(agg, xw, dinv, biases[3], Wl, bl.reshape(1, C))[:N]


# speedup vs baseline: 8.6874x; 1.0056x over previous
import dataclasses

import jax
import jax.numpy as jnp
from jax import lax
from jax.experimental import pallas as pl
from jax.experimental.pallas import tpu as pltpu
from jax.experimental.pallas import tpu_sc as plsc

N = 10000
NP = 10240
E = 320000
F_IN = 128
H = 256
HH = H // 2
C = 40

NSUB = 16
B = 128
BT = 32
ROWS_PER_SUB = NP // NSUB

_mesh = plsc.VectorSubcoreMesh(core_axis_name="c", subcore_axis_name="s")

_cp = pltpu.CompilerParams()
if "needs_layout_passes" in pltpu.CompilerParams.__dataclass_fields__:
    _cp = dataclasses.replace(_cp, needs_layout_passes=False)


def _deg_body(dst_hbm, ew_hbm, out_hbm,
              didx_bufs, ew_bufs, rows_bufs, tdidx_v, zb_v, acc_sh,
              dsems, esems, ssems):
    c = lax.axis_index("c")
    s = lax.axis_index("s")

    @pl.loop(0, 64)
    def _(i):
        for k in range(8):
            zb_v[i, pl.ds(k * 16, 16)] = jnp.zeros((16,), jnp.float32)

    for j in range(ROWS_PER_SUB // 64):
        pltpu.sync_copy(
            zb_v, acc_sh.at[pl.ds(s * ROWS_PER_SUB + j * 64, 64), :])
    plsc.subcore_barrier()

    chunk = E // (2 * NSUB)
    cbase = c * (E // 2) + s * chunk
    nmain = (chunk // B) // NBUF * NBUF

    @pl.loop(0, nmain // NBUF)
    def _(a):
        base = cbase + a * (B * NBUF)
        dw = []
        ewt = []
        for b in range(NBUF):
            bb = base + b * B
            dw.append(pltpu.async_copy(
                dst_hbm.at[pl.ds(bb, B)], didx_bufs[b], dsems[b]))
            ewt.append(pltpu.async_copy(
                ew_hbm.at[pl.ds(bb, B)], ew_bufs[b], esems[b]))
        sw = []
        for b in range(NBUF):
            ewt[b].wait()
            rows_v = rows_bufs[b]
            ew_v = ew_bufs[b]

            @pl.loop(0, B, step=2)
            def _(i):
                for u in range(2):
                    nr = plsc.load_gather(
                        ew_v, [jnp.full((16,), i + u, jnp.int32)])
                    for k in range(8):
                        rows_v[i + u, pl.ds(k * 16, 16)] = nr

            dw[b].wait()
            sw.append(pltpu.async_copy(
                rows_v, acc_sh.at[didx_bufs[b]], ssems[b], add=True))
        for b in range(NBUF):
            sw[b].wait()

    tb = cbase + nmain * B
    tn = chunk - nmain * B
    pltpu.sync_copy(dst_hbm.at[pl.ds(tb, tn)], tdidx_v)
    pltpu.sync_copy(ew_hbm.at[pl.ds(tb, tn)], ew_bufs[0].at[pl.ds(0, tn)])

    @pl.loop(0, tn)
    def _(i):
        nr = plsc.load_gather(ew_bufs[0], [jnp.full((16,), i, jnp.int32)])
        for k in range(8):
            rows_bufs[0][i, pl.ds(k * 16, 16)] = nr

    pltpu.sync_copy(rows_bufs[0].at[pl.ds(0, tn), :],
                    acc_sh.at[tdidx_v], add=True)

    plsc.subcore_barrier()
    pltpu.sync_copy(
        acc_sh.at[pl.ds(s * ROWS_PER_SUB, ROWS_PER_SUB), :],
        out_hbm.at[pl.ds(c * NP + s * ROWS_PER_SUB, ROWS_PER_SUB), :],
    )


def _sc_deg(dst, ew):
    k = pl.kernel(
        _deg_body,
        compiler_params=_cp,
        out_type=jax.ShapeDtypeStruct((2 * NP, HH), jnp.float32),
        mesh=_mesh,
        scratch_types=[
            [pltpu.VMEM((B,), jnp.int32) for _ in range(NBUF)],
            [pltpu.VMEM((B,), jnp.float32) for _ in range(NBUF)],
            [pltpu.VMEM((B, HH), jnp.float32) for _ in range(NBUF)],
            pltpu.VMEM((E // (2 * NSUB) - (E // (2 * NSUB)) // B // NBUF * NBUF * B,), jnp.int32),
            pltpu.VMEM((64, HH), jnp.float32),
            pltpu.VMEM_SHARED((NP, HH), jnp.float32),
            [pltpu.SemaphoreType.DMA for _ in range(NBUF)],
            [pltpu.SemaphoreType.DMA for _ in range(NBUF)],
            [pltpu.SemaphoreType.DMA for _ in range(NBUF)],
        ],
    )
    return k(dst, ew)


def _norm_body(src_hbm, dst_hbm, ew_hbm, dinv_hbm, out_hbm,
               dv_v, sidx_v, didx_v, ew_v, nrm_v):
    c = lax.axis_index("c")
    s = lax.axis_index("s")
    w = s * 2 + c
    pltpu.sync_copy(dinv_hbm, dv_v)

    chunk = E // (2 * NSUB)

    @pl.loop(0, chunk // B)
    def _(t):
        base = w * chunk + t * B
        pltpu.sync_copy(src_hbm.at[pl.ds(base, B)], sidx_v)
        pltpu.sync_copy(dst_hbm.at[pl.ds(base, B)], didx_v)
        pltpu.sync_copy(ew_hbm.at[pl.ds(base, B)], ew_v)
        for j in range(B // 16):
            s16 = sidx_v[pl.ds(j * 16, 16)]
            d16 = didx_v[pl.ds(j * 16, 16)]
            e16 = ew_v[pl.ds(j * 16, 16)]
            g1 = plsc.load_gather(dv_v, [s16])
            g2 = plsc.load_gather(dv_v, [d16])
            nrm_v[pl.ds(j * 16, 16)] = g1 * e16 * g2
        pltpu.sync_copy(nrm_v, out_hbm.at[pl.ds(base, B)])

    nmain = chunk // B
    tn = chunk - nmain * B
    tb = w * chunk + nmain * B
    pltpu.sync_copy(src_hbm.at[pl.ds(tb, tn)], sidx_v.at[pl.ds(0, tn)])
    pltpu.sync_copy(dst_hbm.at[pl.ds(tb, tn)], didx_v.at[pl.ds(0, tn)])
    pltpu.sync_copy(ew_hbm.at[pl.ds(tb, tn)], ew_v.at[pl.ds(0, tn)])
    for j in range(tn // 16):
        s16 = sidx_v[pl.ds(j * 16, 16)]
        d16 = didx_v[pl.ds(j * 16, 16)]
        e16 = ew_v[pl.ds(j * 16, 16)]
        g1 = plsc.load_gather(dv_v, [s16])
        g2 = plsc.load_gather(dv_v, [d16])
        nrm_v[pl.ds(j * 16, 16)] = g1 * e16 * g2
    pltpu.sync_copy(nrm_v.at[pl.ds(0, tn)], out_hbm.at[pl.ds(tb, tn)])


def _sc_norm(src, dst, ew, dinv):
    k = pl.kernel(
        _norm_body,
        compiler_params=_cp,
        out_type=jax.ShapeDtypeStruct((E,), jnp.float32),
        mesh=_mesh,
        scratch_types=[
            pltpu.VMEM((NP,), jnp.float32),
            pltpu.VMEM((B,), jnp.int32),
            pltpu.VMEM((B,), jnp.int32),
            pltpu.VMEM((B,), jnp.float32),
            pltpu.VMEM((B,), jnp.float32),
        ],
    )
    return k(src, dst, ew, dinv)


NBUF = 2


def _agg_body(xw_hbm, src_hbm, dst_hbm, nrm_hbm, out_hbm,
              sidx_bufs, didx_bufs, nrm_bufs, rows_bufs, tdidx_v, zb_v, acc_sh,
              isems, gsems, dsems, nsems, ssems):
    c = lax.axis_index("c")
    s = lax.axis_index("s")

    @pl.loop(0, 64)
    def _(i):
        for k in range(8):
            zb_v[i, pl.ds(k * 16, 16)] = jnp.zeros((16,), jnp.float32)

    for j in range(ROWS_PER_SUB // 64):
        pltpu.sync_copy(
            zb_v, acc_sh.at[pl.ds(s * ROWS_PER_SUB + j * 64, 64), :])
    plsc.subcore_barrier()

    chunk = E // NSUB
    cbase = s * chunk
    roff = jnp.full((16,), c * NP, jnp.int32)

    nmain = (chunk // B) // NBUF * NBUF

    @pl.loop(0, nmain // NBUF)
    def _(a):
        base = cbase + a * (B * NBUF)
        iw = []
        dw = []
        nw = []
        for b in range(NBUF):
            bb = base + b * B
            iw.append(pltpu.async_copy(
                src_hbm.at[pl.ds(bb, B)], sidx_bufs[b], isems[b]))
            dw.append(pltpu.async_copy(
                dst_hbm.at[pl.ds(bb, B)], didx_bufs[b], dsems[b]))
            nw.append(pltpu.async_copy(
                nrm_hbm.at[pl.ds(bb, B)], nrm_bufs[b], nsems[b]))
        gw = []
        for b in range(NBUF):
            iw[b].wait()
            sidx_v = sidx_bufs[b]
            for j in range(B // 16):
                sl = pl.ds(j * 16, 16)
                sidx_v[sl] = sidx_v[sl] + roff
            gw.append(pltpu.async_copy(
                xw_hbm.at[sidx_v], rows_bufs[b], gsems[b]))
        sw = []
        for b in range(NBUF):
            gw[b].wait()
            nw[b].wait()
            rows_v = rows_bufs[b]
            nrm_v = nrm_bufs[b]

            @pl.loop(0, B, step=4)
            def _(i):
                for u in range(4):
                    nr = plsc.load_gather(
                        nrm_v, [jnp.full((16,), i + u, jnp.int32)])
                    for k in range(8):
                        sl = (i + u, pl.ds(k * 16, 16))
                        rows_v[sl] = rows_v[sl] * nr

            dw[b].wait()
            sw.append(pltpu.async_copy(
                rows_v, acc_sh.at[didx_bufs[b]], ssems[b], add=True))
        for b in range(NBUF):
            sw[b].wait()

    tb = cbase + nmain * B
    pltpu.sync_copy(src_hbm.at[pl.ds(tb, BT)], sidx_bufs[0].at[pl.ds(0, BT)])
    pltpu.sync_copy(dst_hbm.at[pl.ds(tb, BT)], tdidx_v)
    pltpu.sync_copy(nrm_hbm.at[pl.ds(tb, BT)], nrm_bufs[0].at[pl.ds(0, BT)])
    for j in range(BT // 16):
        sl = pl.ds(j * 16, 16)
        sidx_bufs[0][sl] = sidx_bufs[0][sl] + roff
    pltpu.sync_copy(xw_hbm.at[sidx_bufs[0].at[pl.ds(0, BT)]],
                    rows_bufs[0].at[pl.ds(0, BT), :])

    @pl.loop(0, BT)
    def _(i):
        nr = plsc.load_gather(nrm_bufs[0], [jnp.full((16,), i, jnp.int32)])
        for k in range(8):
            sl = (i, pl.ds(k * 16, 16))
            rows_bufs[0][sl] = rows_bufs[0][sl] * nr

    pltpu.sync_copy(rows_bufs[0].at[pl.ds(0, BT), :],
                    acc_sh.at[tdidx_v], add=True)

    plsc.subcore_barrier()
    pltpu.sync_copy(
        acc_sh.at[pl.ds(s * ROWS_PER_SUB, ROWS_PER_SUB), :],
        out_hbm.at[pl.ds(c * NP + s * ROWS_PER_SUB, ROWS_PER_SUB), :],
    )


def _sc_agg(xw_flat, src, dst, nrm):
    k = pl.kernel(
        _agg_body,
        compiler_params=_cp,
        out_type=jax.ShapeDtypeStruct((2 * NP, HH), jnp.float32),
        mesh=_mesh,
        scratch_types=[
            [pltpu.VMEM((B,), jnp.int32) for _ in range(NBUF)],
            [pltpu.VMEM((B,), jnp.int32) for _ in range(NBUF)],
            [pltpu.VMEM((B,), jnp.float32) for _ in range(NBUF)],
            [pltpu.VMEM((B, HH), jnp.float32) for _ in range(NBUF)],
            pltpu.VMEM((BT,), jnp.int32),
            pltpu.VMEM((64, HH), jnp.float32),
            pltpu.VMEM_SHARED((NP, HH), jnp.float32),
            [pltpu.SemaphoreType.DMA for _ in range(NBUF)],
            [pltpu.SemaphoreType.DMA for _ in range(NBUF)],
            [pltpu.SemaphoreType.DMA for _ in range(NBUF)],
            [pltpu.SemaphoreType.DMA for _ in range(NBUF)],
            [pltpu.SemaphoreType.DMA for _ in range(NBUF)],
        ],
    )
    return k(xw_flat, src, dst, nrm)


_BLK = 1024


def _dot(a, b):
    return lax.dot_general(a, b, (((1,), (0,)), ((), ())),
                           precision=lax.Precision.HIGHEST,
                           preferred_element_type=jnp.float32)


def _first_body(x_ref, w_ref, deg_ref, xw_ref, dinv_ref):
    deg = deg_ref[0, :, 0] + deg_ref[1, :, 0] + 1.0
    dinv = jnp.where(deg > 0, lax.rsqrt(deg), 0.0)
    dinv_ref[...] = dinv[:, None]
    xw = _dot(x_ref[...], w_ref[...])
    xw_ref[0] = xw[:, :HH]
    xw_ref[1] = xw[:, HH:]


def _tc_first(x, W1, deg2):
    return pl.pallas_call(
        _first_body,
        grid=(NP // _BLK,),
        in_specs=[
            pl.BlockSpec((_BLK, F_IN), lambda i: (i, 0)),
            pl.BlockSpec((F_IN, H), lambda i: (0, 0)),
            pl.BlockSpec((2, _BLK, HH), lambda i: (0, i, 0)),
        ],
        out_specs=[
            pl.BlockSpec((2, _BLK, HH), lambda i: (0, i, 0)),
            pl.BlockSpec((_BLK, 1), lambda i: (i, 0)),
        ],
        out_shape=[
            jax.ShapeDtypeStruct((2, NP, HH), jnp.float32),
            jax.ShapeDtypeStruct((NP, 1), jnp.float32),
        ],
    )(x, W1, deg2)


def _mid_body(agg_ref, xwp_ref, dinv_ref, b_ref, w_ref, out_ref):
    d2 = dinv_ref[...] * dinv_ref[...]
    t = jnp.concatenate(
        [agg_ref[0] + xwp_ref[0] * d2, agg_ref[1] + xwp_ref[1] * d2], axis=1)
    t = t + b_ref[...]
    nrm = jnp.sqrt(jnp.sum(t * t, axis=1, keepdims=True))
    r = t / jnp.maximum(nrm, 1e-12)
    r = jnp.maximum(r, 0.0)
    xw = _dot(r, w_ref[...])
    out_ref[0] = xw[:, :HH]
    out_ref[1] = xw[:, HH:]


def _tc_mid(agg, xwp, dinv, b, Wn):
    return pl.pallas_call(
        _mid_body,
        grid=(NP // _BLK,),
        in_specs=[
            pl.BlockSpec((2, _BLK, HH), lambda i: (0, i, 0)),
            pl.BlockSpec((2, _BLK, HH), lambda i: (0, i, 0)),
            pl.BlockSpec((_BLK, 1), lambda i: (i, 0)),
            pl.BlockSpec((1, H), lambda i: (0, 0)),
            pl.BlockSpec((H, H), lambda i: (0, 0)),
        ],
        out_specs=pl.BlockSpec((2, _BLK, HH), lambda i: (0, i, 0)),
        out_shape=jax.ShapeDtypeStruct((2, NP, HH), jnp.float32),
    )(agg, xwp, dinv, b, Wn)


def _last_body(agg_ref, xwp_ref, dinv_ref, b_ref, wl_ref, bl_ref, out_ref):
    d2 = dinv_ref[...] * dinv_ref[...]
    t = jnp.concatenate(
        [agg_ref[0] + xwp_ref[0] * d2, agg_ref[1] + xwp_ref[1] * d2], axis=1)
    t = t + b_ref[...]
    nrm = jnp.sqrt(jnp.sum(t * t, axis=1, keepdims=True))
    r = t / jnp.maximum(nrm, 1e-12)
    r = jnp.maximum(r, 0.0)
    out_ref[...] = _dot(r, wl_ref[...]) + bl_ref[...]


def _tc_last(agg, xwp, dinv, b, Wl, bl):
    return pl.pallas_call(
        _last_body,
        grid=(NP // _BLK,),
        in_specs=[
            pl.BlockSpec((2, _BLK, HH), lambda i: (0, i, 0)),
            pl.BlockSpec((2, _BLK, HH), lambda i: (0, i, 0)),
            pl.BlockSpec((_BLK, 1), lambda i: (i, 0)),
            pl.BlockSpec((1, H), lambda i: (0, 0)),
            pl.BlockSpec((H, C), lambda i: (0, 0)),
            pl.BlockSpec((1, C), lambda i: (0, 0)),
        ],
        out_specs=pl.BlockSpec((_BLK, C), lambda i: (i, 0)),
        out_shape=jax.ShapeDtypeStruct((NP, C), jnp.float32),
    )(agg, xwp, dinv, b, Wl, bl)


def kernel(x, edge_index, edge_weights, W1, b1, W2, b2, W3, b3, W4, b4, Wl, bl):
    src = edge_index[0]
    dst = edge_index[1]
    ew = edge_weights.astype(jnp.float32)

    deg2 = _sc_deg(dst, ew).reshape(2, NP, HH)
    xp = jnp.pad(x, ((0, NP - N), (0, 0)))
    xw, dinv = _tc_first(xp, W1, deg2)
    nrm = _sc_norm(src, dst, ew, dinv.reshape(NP))

    biases = [b1.reshape(1, H), b2.reshape(1, H), b3.reshape(1, H),
              b4.reshape(1, H)]
    weights = [W2, W3, W4]

    for l in range(3):
        agg = _sc_agg(xw.reshape(2 * NP, HH), src, dst, nrm).reshape(2, NP, HH)
        xw = _tc_mid(agg, xw, dinv, biases[l], weights[l])

    agg = _sc_agg(xw.reshape(2 * NP, HH), src, dst, nrm).reshape(2, NP, HH)
    return _tc_last(agg, xw, dinv, biases[3], Wl, bl.reshape(1, C))[:N]
